# Initial kernel scaffold; baseline (speedup 1.0000x reference)
import jax
import jax.numpy as jnp
from jax.experimental import pallas as pl


def kernel(x, depth_table, height_table, width_table):
    raise NotImplementedError("write your pallas kernel here")


# speedup vs baseline: 2.5534x; 2.5534x over previous
import functools

import jax
import jax.numpy as jnp
from jax.experimental import pallas as pl
from jax.experimental.pallas import tpu as pltpu

DB = 8


def _tc_body(dt_ref, x_ref, ht_ref, wt_ref, out_ref):
    c = pl.program_id(1)
    d0 = pl.program_id(2) * DB
    hw = ht_ref[0] + wt_ref[0]
    for dl in range(DB):
        s = dt_ref[d0 + dl, c]
        out_ref[0, 0, dl] = x_ref[0, 0, dl] + (hw + s)


@jax.jit
def kernel(x, depth_table, height_table, width_table):
    B, C, D, H, W = x.shape
    ht_t = height_table.T.reshape(C, H, 1)
    wt_t = width_table.T.reshape(C, 1, W)

    grid = (B, C, D // DB)
    return pl.pallas_call(
        _tc_body,
        grid=grid,
        in_specs=[
            pl.BlockSpec(memory_space=pltpu.SMEM),
            pl.BlockSpec((1, 1, DB, H, W), lambda b, c, g: (b, c, g, 0, 0)),
            pl.BlockSpec((1, H, 1), lambda b, c, g: (c, 0, 0)),
            pl.BlockSpec((1, 1, W), lambda b, c, g: (c, 0, 0)),
        ],
        out_specs=pl.BlockSpec((1, 1, DB, H, W), lambda b, c, g: (b, c, g, 0, 0)),
        out_shape=jax.ShapeDtypeStruct(x.shape, x.dtype),
    )(depth_table, x, ht_t, wt_t)


# speedup vs baseline: 3.9436x; 1.5444x over previous
import functools

import jax
import jax.numpy as jnp
from jax.experimental import pallas as pl
from jax.experimental.pallas import tpu as pltpu

DB = 32


def _tc_body(dt_ref, x_ref, ht_ref, wt_ref, out_ref):
    c = pl.program_id(1)
    d0 = pl.program_id(2) * DB
    hw = ht_ref[0] + wt_ref[0]
    for dl in range(DB):
        s = dt_ref[d0 + dl, c]
        out_ref[0, 0, dl] = x_ref[0, 0, dl] + (hw + s)


@jax.jit
def kernel(x, depth_table, height_table, width_table):
    B, C, D, H, W = x.shape
    ht_t = height_table.T.reshape(C, H, 1)
    wt_t = width_table.T.reshape(C, 1, W)

    grid = (B, C, D // DB)
    return pl.pallas_call(
        _tc_body,
        grid=grid,
        in_specs=[
            pl.BlockSpec(memory_space=pltpu.SMEM),
            pl.BlockSpec((1, 1, DB, H, W), lambda b, c, g: (b, c, g, 0, 0)),
            pl.BlockSpec((1, H, 1), lambda b, c, g: (c, 0, 0)),
            pl.BlockSpec((1, 1, W), lambda b, c, g: (c, 0, 0)),
        ],
        out_specs=pl.BlockSpec((1, 1, DB, H, W), lambda b, c, g: (b, c, g, 0, 0)),
        out_shape=jax.ShapeDtypeStruct(x.shape, x.dtype),
    )(depth_table, x, ht_t, wt_t)


# speedup vs baseline: 6.3068x; 1.5992x over previous
import jax
import jax.numpy as jnp
from jax import lax
from jax.experimental import pallas as pl
from jax.experimental.pallas import tpu as pltpu

DB = 32


def _tc_body(dt_ref, x_ref, he_ref, ho_ref, wt_ref, out_ref):
    c = pl.program_id(1)
    d0 = pl.program_id(2) * DB
    HR, W = he_ref.shape[1], wt_ref.shape[2]
    lane = lax.broadcasted_iota(jnp.int32, (HR, 2 * W), 1)
    hterm = jnp.where(lane < W, he_ref[0], ho_ref[0])
    w2 = jnp.concatenate([wt_ref[0], wt_ref[0]], axis=-1)
    hw = hterm + w2
    for dl in range(DB):
        s = dt_ref[d0 + dl, c]
        out_ref[0, 0, dl] = x_ref[0, 0, dl] + (hw + s)


@jax.jit
def kernel(x, depth_table, height_table, width_table):
    B, C, D, H, W = x.shape
    xp = x.reshape(B, C, D, H // 2, 2 * W)
    ht_t = height_table.T
    he = ht_t[:, 0::2].reshape(C, H // 2, 1)
    ho = ht_t[:, 1::2].reshape(C, H // 2, 1)
    wt_t = width_table.T.reshape(C, 1, W)

    grid = (B, C, D // DB)
    out = pl.pallas_call(
        _tc_body,
        grid=grid,
        in_specs=[
            pl.BlockSpec(memory_space=pltpu.SMEM),
            pl.BlockSpec((1, 1, DB, H // 2, 2 * W), lambda b, c, g: (b, c, g, 0, 0)),
            pl.BlockSpec((1, H // 2, 1), lambda b, c, g: (c, 0, 0)),
            pl.BlockSpec((1, H // 2, 1), lambda b, c, g: (c, 0, 0)),
            pl.BlockSpec((1, 1, W), lambda b, c, g: (c, 0, 0)),
        ],
        out_specs=pl.BlockSpec((1, 1, DB, H // 2, 2 * W), lambda b, c, g: (b, c, g, 0, 0)),
        out_shape=jax.ShapeDtypeStruct(xp.shape, x.dtype),
    )(depth_table, xp, he, ho, wt_t)
    return out.reshape(B, C, D, H, W)
